# baseline (device time: 22621 ns/iter reference)
import jax
import jax.numpy as jnp
from jax import lax
from jax.experimental import pallas as pl
from jax.experimental.pallas import tpu as pltpu

N_DEV = 16


def kernel(A, B):
    m, k_per = A.shape
    _, n = B.shape
    m_per = m // N_DEV

    def body(a_ref, b_ref, out_ref, partial_ref, comm_ref, send_sems, recv_sems):
        my = lax.axis_index("i")

        partial_ref[:, :] = jnp.dot(
            a_ref[:, :], b_ref[:, :], preferred_element_type=jnp.float32
        )

        sends = []
        for d in range(1, N_DEV):
            t = lax.rem(my + d, N_DEV)
            rdma = pltpu.make_async_remote_copy(
                src_ref=partial_ref.at[pl.ds(t * m_per, m_per), :],
                dst_ref=comm_ref.at[my],
                send_sem=send_sems.at[t],
                recv_sem=recv_sems.at[my],
                device_id=(t,),
                device_id_type=pl.DeviceIdType.MESH,
            )
            rdma.start()
            sends.append(rdma)

        comm_ref[pl.ds(my, 1), :, :] = partial_ref[pl.ds(my * m_per, m_per), :][None]

        for d in range(1, N_DEV):
            s = lax.rem(my + d, N_DEV)
            recv = pltpu.make_async_remote_copy(
                src_ref=partial_ref.at[pl.ds(0, m_per), :],
                dst_ref=comm_ref.at[s],
                send_sem=send_sems.at[s],
                recv_sem=recv_sems.at[s],
                device_id=(s,),
                device_id_type=pl.DeviceIdType.MESH,
            )
            recv.wait_recv()

        out_ref[:, :] = jnp.sum(comm_ref[:, :, :], axis=0)

        for rdma in sends:
            rdma.wait_send()

    return pl.pallas_call(
        body,
        out_shape=jax.ShapeDtypeStruct((m_per, n), jnp.float32),
        in_specs=[
            pl.BlockSpec(memory_space=pltpu.VMEM),
            pl.BlockSpec(memory_space=pltpu.VMEM),
        ],
        out_specs=pl.BlockSpec(memory_space=pltpu.VMEM),
        scratch_shapes=[
            pltpu.VMEM((m, n), jnp.float32),
            pltpu.VMEM((N_DEV, m_per, n), jnp.float32),
            pltpu.SemaphoreType.DMA((N_DEV,)),
            pltpu.SemaphoreType.DMA((N_DEV,)),
        ],
    )(A, B)


# device time: 14147 ns/iter; 1.5990x vs baseline; 1.5990x over previous
import jax
import jax.numpy as jnp
from jax import lax
from jax.experimental import pallas as pl
from jax.experimental.pallas import tpu as pltpu

N_DEV = 16


def kernel(A, B):
    m, k_per = A.shape
    _, n = B.shape
    m_per = m // N_DEV

    def body(a_ref, b_ref, out_ref, partial_ref, comm_ref, send_sems, recv_sems):
        my = lax.axis_index("i")

        barrier_sem = pltpu.get_barrier_semaphore()
        for d in range(1, N_DEV):
            t = lax.rem(my + d, N_DEV)
            pl.semaphore_signal(
                barrier_sem, inc=1,
                device_id=(t,), device_id_type=pl.DeviceIdType.MESH,
            )

        partial_ref[:, :] = jnp.dot(
            a_ref[:, :], b_ref[:, :], preferred_element_type=jnp.float32
        ).astype(jnp.bfloat16)

        pl.semaphore_wait(barrier_sem, N_DEV - 1)

        sends = []
        for d in range(1, N_DEV):
            t = lax.rem(my + d, N_DEV)
            rdma = pltpu.make_async_remote_copy(
                src_ref=partial_ref.at[pl.ds(t * m_per, m_per), :],
                dst_ref=comm_ref.at[my],
                send_sem=send_sems.at[t],
                recv_sem=recv_sems.at[my],
                device_id=(t,),
                device_id_type=pl.DeviceIdType.MESH,
            )
            rdma.start()
            sends.append(rdma)

        comm_ref[pl.ds(my, 1), :, :] = partial_ref[pl.ds(my * m_per, m_per), :][None]

        for d in range(1, N_DEV):
            s = lax.rem(my + d, N_DEV)
            recv = pltpu.make_async_remote_copy(
                src_ref=partial_ref.at[pl.ds(0, m_per), :],
                dst_ref=comm_ref.at[s],
                send_sem=send_sems.at[s],
                recv_sem=recv_sems.at[s],
                device_id=(s,),
                device_id_type=pl.DeviceIdType.MESH,
            )
            recv.wait_recv()

        out_ref[:, :] = jnp.sum(
            comm_ref[:, :, :].astype(jnp.float32), axis=0
        )

        for rdma in sends:
            rdma.wait_send()

    return pl.pallas_call(
        body,
        out_shape=jax.ShapeDtypeStruct((m_per, n), jnp.float32),
        in_specs=[
            pl.BlockSpec(memory_space=pltpu.VMEM),
            pl.BlockSpec(memory_space=pltpu.VMEM),
        ],
        out_specs=pl.BlockSpec(memory_space=pltpu.VMEM),
        scratch_shapes=[
            pltpu.VMEM((m, n), jnp.bfloat16),
            pltpu.VMEM((N_DEV, m_per, n), jnp.bfloat16),
            pltpu.SemaphoreType.DMA((N_DEV,)),
            pltpu.SemaphoreType.DMA((N_DEV,)),
        ],
        compiler_params=pltpu.CompilerParams(collective_id=0),
    )(A, B)


# device time: 14144 ns/iter; 1.5993x vs baseline; 1.0002x over previous
import jax
import jax.numpy as jnp
from jax import lax
from jax.experimental import pallas as pl
from jax.experimental.pallas import tpu as pltpu

N_DEV = 16


def kernel(A, B):
    m, k_per = A.shape
    _, n = B.shape
    m_per = m // N_DEV

    def body(a_ref, b_ref, out_ref, partial_ref, comm_ref, send_sems, recv_sems):
        my = lax.axis_index("i")

        barrier_sem = pltpu.get_barrier_semaphore()
        for d in range(1, N_DEV):
            t = lax.rem(my + d, N_DEV)
            pl.semaphore_signal(
                barrier_sem, inc=1,
                device_id=(t,), device_id_type=pl.DeviceIdType.MESH,
            )

        partial_ref[:, :] = jnp.dot(
            a_ref[:, :].astype(jnp.bfloat16),
            b_ref[:, :].astype(jnp.bfloat16),
            preferred_element_type=jnp.float32,
        ).astype(jnp.bfloat16)

        pl.semaphore_wait(barrier_sem, N_DEV - 1)

        sends = []
        for d in range(1, N_DEV):
            t = lax.rem(my + d, N_DEV)
            rdma = pltpu.make_async_remote_copy(
                src_ref=partial_ref.at[pl.ds(t * m_per, m_per), :],
                dst_ref=comm_ref.at[N_DEV - d],
                send_sem=send_sems.at[d],
                recv_sem=recv_sems.at[N_DEV - d],
                device_id=(t,),
                device_id_type=pl.DeviceIdType.MESH,
            )
            rdma.start()
            sends.append(rdma)

        comm_ref[0, :, :] = partial_ref[pl.ds(my * m_per, m_per), :]

        def wait_slot(j):
            recv = pltpu.make_async_remote_copy(
                src_ref=partial_ref.at[pl.ds(0, m_per), :],
                dst_ref=comm_ref.at[j],
                send_sem=send_sems.at[j],
                recv_sem=recv_sems.at[j],
                device_id=(my,),
                device_id_type=pl.DeviceIdType.MESH,
            )
            recv.wait_recv()

        half = N_DEV // 2
        for j in range(1, half):
            wait_slot(j)
        acc = jnp.sum(comm_ref[0:half, :, :].astype(jnp.float32), axis=0)

        for j in range(half, N_DEV):
            wait_slot(j)
        out_ref[:, :] = acc + jnp.sum(
            comm_ref[half:N_DEV, :, :].astype(jnp.float32), axis=0
        )

        for rdma in sends:
            rdma.wait_send()

    return pl.pallas_call(
        body,
        out_shape=jax.ShapeDtypeStruct((m_per, n), jnp.float32),
        in_specs=[
            pl.BlockSpec(memory_space=pltpu.VMEM),
            pl.BlockSpec(memory_space=pltpu.VMEM),
        ],
        out_specs=pl.BlockSpec(memory_space=pltpu.VMEM),
        scratch_shapes=[
            pltpu.VMEM((m, n), jnp.bfloat16),
            pltpu.VMEM((N_DEV, m_per, n), jnp.bfloat16),
            pltpu.SemaphoreType.DMA((N_DEV,)),
            pltpu.SemaphoreType.DMA((N_DEV,)),
        ],
        compiler_params=pltpu.CompilerParams(collective_id=0),
    )(A, B)
